# baseline (speedup 1.0000x reference)
import functools
import jax
import jax.numpy as jnp
from jax import lax
from jax.experimental import pallas as pl
from jax.experimental.pallas import tpu as pltpu
from jax.experimental.pallas import tpu_sc as plsc

_NC, _NS, _LANES = 2, 16, 16
_NW = _NC * _NS
_D = 128
_T = 128
_G = 16


def _sc_body(coords_hbm, at_hbm, rt_hbm, mt_hbm, W_hbm, b_hbm,
             atab_hbm, rtab_hbm, mtab_hbm, out_hbm,
             idxa_v, idxr_v, idxm_v, coords_v,
             atab_v, rtab_v, mtab_v, out0, out1, W_v, b_v,
             so0, so1, n_tok):
    pw = n_tok // _NW
    nch = pw // _T
    wid = lax.axis_index("s") * _NC + lax.axis_index("c")
    base = wid * pw

    outs = [out0, out1]
    osems = [so0, so1]

    pltpu.sync_copy(W_hbm, W_v)
    pltpu.sync_copy(b_hbm, b_v)
    pltpu.sync_copy(atab_hbm, atab_v)
    pltpu.sync_copy(rtab_hbm, rtab_v)
    pltpu.sync_copy(mtab_hbm, mtab_v)
    pltpu.sync_copy(at_hbm.at[pl.ds(base, pw)], idxa_v)
    pltpu.sync_copy(rt_hbm.at[pl.ds(base, pw)], idxr_v)
    pltpu.sync_copy(mt_hbm.at[pl.ds(base, pw)], idxm_v)
    pltpu.sync_copy(coords_hbm.at[pl.ds(base * 3, pw * 3)],
                    coords_v.at[pl.ds(0, pw * 3)])
    Wc = [[W_v[pl.ds(c * _D + 16 * k, 16)] for k in range(8)] for c in range(3)]
    bc = [b_v[pl.ds(16 * k, 16)] for k in range(8)]

    def pair_body(cp, carry):
        for b2 in (0, 1):
            ci = cp * 2 + b2
            cb = base + ci * _T
            ct = ci * _T

            @pl.when(cp > 0)
            def _wait_out():
                pltpu.make_async_copy(outs[b2],
                                      out_hbm.at[pl.ds(cb, _T)],
                                      osems[b2]).wait()

            ov = outs[b2]

            @plsc.parallel_loop(0, _T // _G, 1, unroll=1)
            def grp_body(g):
                g16 = ct + g * _G
                iva = idxa_v[pl.ds(g16, _G)] * _D
                ivr = idxr_v[pl.ds(g16, _G)] * _D
                ivm = idxm_v[pl.ds(g16, _G)] * _D
                for u in range(_G):
                    t = g * _G + u
                    ab = iva[u]
                    rb = ivr[u]
                    mb = ivm[u]
                    v = coords_v[pl.ds((g16 + u) * 3, _LANES)]
                    bx = v[0]
                    by = v[1]
                    bz = v[2]
                    for k in range(8):
                        sl = pl.ds(16 * k, 16)
                        pr = (bc[k] + bx * Wc[0][k] + by * Wc[1][k]
                              + bz * Wc[2][k])
                        h = pr / (1.0 + jnp.exp(-pr))
                        vv = (h + atab_v[pl.ds(ab + 16 * k, 16)]
                              + rtab_v[pl.ds(rb + 16 * k, 16)]
                              + mtab_v[pl.ds(mb + 16 * k, 16)])
                        ov[t, sl] = vv

            pltpu.async_copy(ov, out_hbm.at[pl.ds(cb, _T)], osems[b2])
        return carry

    lax.fori_loop(0, nch // 2, pair_body, 0)

    for b2 in (0, 1):
        cb = base + (nch - 2 + b2) * _T
        pltpu.make_async_copy(outs[b2], out_hbm.at[pl.ds(cb, _T)],
                              osems[b2]).wait()


def kernel(coords, atom_types, residue_types, meta_classes, W_coord, b_coord,
           atom_table, residue_table, meta_table):
    B, L, D = coords.shape[0], coords.shape[1], W_coord.shape[1]
    N = B * L
    pw = N // _NW
    coords_f = coords.reshape(N * 3)
    at = atom_types.reshape(N)
    rt = residue_types.reshape(N)
    mt = meta_classes.reshape(N)
    W_f = W_coord.reshape(3 * D)
    atab_f = atom_table.reshape(128 * D)
    rtab_f = residue_table.reshape(32 * D)
    mtab_f = meta_table.reshape(16 * D)

    mesh = plsc.VectorSubcoreMesh(core_axis_name="c", subcore_axis_name="s",
                                  num_cores=_NC, num_subcores=_NS)
    sc_fn = pl.kernel(
        functools.partial(_sc_body, n_tok=N),
        out_type=jax.ShapeDtypeStruct((N, _D), jnp.float32),
        mesh=mesh,
        scratch_types=[
            pltpu.VMEM((pw,), jnp.int32),
            pltpu.VMEM((pw,), jnp.int32),
            pltpu.VMEM((pw,), jnp.int32),
            pltpu.VMEM((pw * 3 + _LANES,), jnp.float32),
            pltpu.VMEM((128 * _D,), jnp.float32),
            pltpu.VMEM((32 * _D,), jnp.float32),
            pltpu.VMEM((16 * _D,), jnp.float32),
            pltpu.VMEM((_T, _D), jnp.float32),
            pltpu.VMEM((_T, _D), jnp.float32),
            pltpu.VMEM((3 * _D,), jnp.float32),
            pltpu.VMEM((_D,), jnp.float32),
            pltpu.SemaphoreType.DMA,
            pltpu.SemaphoreType.DMA,
        ],
    )
    out = sc_fn(coords_f, at, rt, mt, W_f, b_coord,
                atab_f, rtab_f, mtab_f)
    return out.reshape(B, L, D)


# speedup vs baseline: 3.2361x; 3.2361x over previous
import functools
import jax
import jax.numpy as jnp
from jax import lax
from jax.experimental import pallas as pl
from jax.experimental.pallas import tpu as pltpu
from jax.experimental.pallas import tpu_sc as plsc

_NC, _NS, _LANES = 2, 16, 16
_NW = _NC * _NS
_D = 128
_T = 128
_G = 16

_GDN = lax.GatherDimensionNumbers(offset_dims=(), collapsed_slice_dims=(0,),
                                  start_index_map=(0,))


def _bcast_lane(v, lane):
    gi = jnp.full((_LANES, 1), lane, jnp.int32)
    return lax.gather(v, gi, _GDN, (1,),
                      mode=lax.GatherScatterMode.PROMISE_IN_BOUNDS)


def _sc_body(coords_hbm, ia_hbm, ic_hbm, W_hbm, b_hbm,
             atab_hbm, ctab_hbm, out_hbm,
             idxa_v, idxc_v, coords_v,
             atab_v, ctab_v, out0, out1, W_v, b_v,
             so0, so1, n_tok):
    pw = n_tok // _NW
    nch = pw // _T
    wid = lax.axis_index("s") * _NC + lax.axis_index("c")
    base = wid * pw

    outs = [out0, out1]
    osems = [so0, so1]

    pltpu.sync_copy(W_hbm, W_v)
    pltpu.sync_copy(b_hbm, b_v)
    pltpu.sync_copy(atab_hbm, atab_v)
    pltpu.sync_copy(ctab_hbm, ctab_v)
    pltpu.sync_copy(ia_hbm.at[pl.ds(base, pw)], idxa_v)
    pltpu.sync_copy(ic_hbm.at[pl.ds(base, pw)], idxc_v)
    pltpu.sync_copy(coords_hbm.at[pl.ds(base * 3, pw * 3)],
                    coords_v.at[pl.ds(0, pw * 3)])
    Wc = [[W_v[pl.ds(c * _D + 16 * k, 16)] for k in range(8)] for c in range(3)]
    bc = [b_v[pl.ds(16 * k, 16)] for k in range(8)]

    def pair_body(cp, carry):
        for b2 in (0, 1):
            ci = cp * 2 + b2
            ct = ci * _T
            cb = base + ct
            cbase3 = ct * 3

            @pl.when(cp > 0)
            def _wait_out():
                pltpu.make_async_copy(outs[b2],
                                      out_hbm.at[pl.ds(cb, _T)],
                                      osems[b2]).wait()

            ov = outs[b2]

            @plsc.parallel_loop(0, _T, 1, unroll=2)
            def proj_body(t):
                v = coords_v[pl.ds(cbase3 + 3 * t, _LANES)]
                bx = _bcast_lane(v, 0)
                by = _bcast_lane(v, 1)
                bz = _bcast_lane(v, 2)
                for k in range(8):
                    pr = (bc[k] + bx * Wc[0][k] + by * Wc[1][k]
                          + bz * Wc[2][k])
                    h = pr / (1.0 + jnp.exp(-pr))
                    ov[t, pl.ds(16 * k, 16)] = h

            @plsc.parallel_loop(0, _T // _G, 1, unroll=2)
            def rows_body(g):
                g16 = ct + g * _G
                iva = idxa_v[pl.ds(g16, _G)] * _D
                ivc = idxc_v[pl.ds(g16, _G)] * _D
                for u in range(_G):
                    t = g * _G + u
                    ab = iva[u]
                    cbx = ivc[u]
                    la = [atab_v[pl.ds(ab + 16 * k, 16)] for k in range(8)]
                    lc = [ctab_v[pl.ds(cbx + 16 * k, 16)] for k in range(8)]
                    for k in range(8):
                        plsc.addupdate(ov.at[t, pl.ds(16 * k, 16)],
                                       la[k] + lc[k])

            pltpu.async_copy(ov, out_hbm.at[pl.ds(cb, _T)], osems[b2])
        return carry

    lax.fori_loop(0, nch // 2, pair_body, 0)

    for b2 in (0, 1):
        cb = base + (nch - 2 + b2) * _T
        pltpu.make_async_copy(outs[b2], out_hbm.at[pl.ds(cb, _T)],
                              osems[b2]).wait()


def kernel(coords, atom_types, residue_types, meta_classes, W_coord, b_coord,
           atom_table, residue_table, meta_table):
    B, L, D = coords.shape[0], coords.shape[1], W_coord.shape[1]
    N = B * L
    pw = N // _NW
    coords_f = coords.reshape(N * 3)
    ia = atom_types.reshape(N)
    ic = (residue_types * 16 + meta_classes).reshape(N)
    W_f = W_coord.reshape(3 * D)
    atab_f = atom_table.reshape(128 * D)
    ctab_f = (residue_table[:, None, :] + meta_table[None, :, :]
              ).reshape(512 * D)

    mesh = plsc.VectorSubcoreMesh(core_axis_name="c", subcore_axis_name="s",
                                  num_cores=_NC, num_subcores=_NS)
    sc_fn = pl.kernel(
        functools.partial(_sc_body, n_tok=N),
        out_type=jax.ShapeDtypeStruct((N, _D), jnp.float32),
        mesh=mesh,
        scratch_types=[
            pltpu.VMEM((pw,), jnp.int32),
            pltpu.VMEM((pw,), jnp.int32),
            pltpu.VMEM((pw * 3 + _LANES,), jnp.float32),
            pltpu.VMEM((128 * _D,), jnp.float32),
            pltpu.VMEM((512 * _D,), jnp.float32),
            pltpu.VMEM((_T, _D), jnp.float32),
            pltpu.VMEM((_T, _D), jnp.float32),
            pltpu.VMEM((3 * _D,), jnp.float32),
            pltpu.VMEM((_D,), jnp.float32),
            pltpu.SemaphoreType.DMA,
            pltpu.SemaphoreType.DMA,
        ],
    )
    out = sc_fn(coords_f, ia, ic, W_f, b_coord, atab_f, ctab_f)
    return out.reshape(B, L, D)


# speedup vs baseline: 3.2533x; 1.0053x over previous
import functools
import jax
import jax.numpy as jnp
from jax import lax
from jax.experimental import pallas as pl
from jax.experimental.pallas import tpu as pltpu
from jax.experimental.pallas import tpu_sc as plsc

_NC, _NS, _LANES = 2, 16, 16
_NW = _NC * _NS
_D = 128
_T = 128
_G = 16

_GDN = lax.GatherDimensionNumbers(offset_dims=(), collapsed_slice_dims=(0,),
                                  start_index_map=(0,))


def _bcast_lane(v, lane):
    gi = jnp.full((_LANES, 1), lane, jnp.int32)
    return lax.gather(v, gi, _GDN, (1,),
                      mode=lax.GatherScatterMode.PROMISE_IN_BOUNDS)


def _sc_body(coords_hbm, ia_hbm, ic_hbm, W_hbm, b_hbm,
             atab_hbm, ctab_hbm, out_hbm,
             idxa_v, idxc_v, coords_v,
             atab_v, ctab_v, out0, out1, W_v, b_v,
             so0, so1, n_tok):
    pw = n_tok // _NW
    nch = pw // _T
    wid = lax.axis_index("s") * _NC + lax.axis_index("c")
    base = wid * pw

    outs = [out0, out1]
    osems = [so0, so1]

    pro = [
        pltpu.async_copy(W_hbm, W_v, so0),
        pltpu.async_copy(b_hbm, b_v, so0),
        pltpu.async_copy(atab_hbm, atab_v, so0),
        pltpu.async_copy(ctab_hbm, ctab_v, so0),
        pltpu.async_copy(ia_hbm.at[pl.ds(base, pw)], idxa_v, so0),
        pltpu.async_copy(ic_hbm.at[pl.ds(base, pw)], idxc_v, so0),
        pltpu.async_copy(coords_hbm.at[pl.ds(base * 3, pw * 3)],
                         coords_v.at[pl.ds(0, pw * 3)], so0),
    ]
    for c in pro:
        c.wait()
    Wc = [[W_v[pl.ds(c * _D + 16 * k, 16)] for k in range(8)] for c in range(3)]
    bc = [b_v[pl.ds(16 * k, 16)] for k in range(8)]

    def pair_body(cp, carry):
        for b2 in (0, 1):
            ci = cp * 2 + b2
            ct = ci * _T
            cb = base + ct
            cbase3 = ct * 3

            @pl.when(cp > 0)
            def _wait_out():
                pltpu.make_async_copy(outs[b2],
                                      out_hbm.at[pl.ds(cb, _T)],
                                      osems[b2]).wait()

            ov = outs[b2]

            @plsc.parallel_loop(0, _T, 1, unroll=2)
            def proj_body(t):
                v = coords_v[pl.ds(cbase3 + 3 * t, _LANES)]
                bx = _bcast_lane(v, 0)
                by = _bcast_lane(v, 1)
                bz = _bcast_lane(v, 2)
                for k in range(8):
                    pr = (bc[k] + bx * Wc[0][k] + by * Wc[1][k]
                          + bz * Wc[2][k])
                    h = pr / (1.0 + jnp.exp(-pr))
                    ov[t, pl.ds(16 * k, 16)] = h

            @plsc.parallel_loop(0, _T // _G, 1, unroll=2)
            def rows_body(g):
                g16 = ct + g * _G
                iva = idxa_v[pl.ds(g16, _G)] * _D
                ivc = idxc_v[pl.ds(g16, _G)] * _D
                for u in range(_G):
                    t = g * _G + u
                    ab = iva[u]
                    cbx = ivc[u]
                    la = [atab_v[pl.ds(ab + 16 * k, 16)] for k in range(8)]
                    lc = [ctab_v[pl.ds(cbx + 16 * k, 16)] for k in range(8)]
                    for k in range(8):
                        plsc.addupdate(ov.at[t, pl.ds(16 * k, 16)],
                                       la[k] + lc[k])

            pltpu.async_copy(ov, out_hbm.at[pl.ds(cb, _T)], osems[b2])
        return carry

    lax.fori_loop(0, nch // 2, pair_body, 0)

    for b2 in (0, 1):
        cb = base + (nch - 2 + b2) * _T
        pltpu.make_async_copy(outs[b2], out_hbm.at[pl.ds(cb, _T)],
                              osems[b2]).wait()


def kernel(coords, atom_types, residue_types, meta_classes, W_coord, b_coord,
           atom_table, residue_table, meta_table):
    B, L, D = coords.shape[0], coords.shape[1], W_coord.shape[1]
    N = B * L
    pw = N // _NW
    coords_f = coords.reshape(N * 3)
    ia = atom_types.reshape(N)
    ic = (residue_types * 16 + meta_classes).reshape(N)
    W_f = W_coord.reshape(3 * D)
    atab_f = atom_table.reshape(128 * D)
    ctab_f = (residue_table[:, None, :] + meta_table[None, :, :]
              ).reshape(512 * D)

    mesh = plsc.VectorSubcoreMesh(core_axis_name="c", subcore_axis_name="s",
                                  num_cores=_NC, num_subcores=_NS)
    sc_fn = pl.kernel(
        functools.partial(_sc_body, n_tok=N),
        out_type=jax.ShapeDtypeStruct((N, _D), jnp.float32),
        mesh=mesh,
        scratch_types=[
            pltpu.VMEM((pw,), jnp.int32),
            pltpu.VMEM((pw,), jnp.int32),
            pltpu.VMEM((pw * 3 + _LANES,), jnp.float32),
            pltpu.VMEM((128 * _D,), jnp.float32),
            pltpu.VMEM((512 * _D,), jnp.float32),
            pltpu.VMEM((_T, _D), jnp.float32),
            pltpu.VMEM((_T, _D), jnp.float32),
            pltpu.VMEM((3 * _D,), jnp.float32),
            pltpu.VMEM((_D,), jnp.float32),
            pltpu.SemaphoreType.DMA,
            pltpu.SemaphoreType.DMA,
        ],
    )
    out = sc_fn(coords_f, ia, ic, W_f, b_coord, atab_f, ctab_f)
    return out.reshape(B, L, D)


# speedup vs baseline: 3.2721x; 1.0058x over previous
import functools
import jax
import jax.numpy as jnp
from jax import lax
from jax.experimental import pallas as pl
from jax.experimental.pallas import tpu as pltpu
from jax.experimental.pallas import tpu_sc as plsc

_NC, _NS, _LANES = 2, 16, 16
_NW = _NC * _NS
_D = 128
_T = 128
_G = 16

_GDN = lax.GatherDimensionNumbers(offset_dims=(), collapsed_slice_dims=(0,),
                                  start_index_map=(0,))


def _bcast_lane(v, lane):
    gi = jnp.full((_LANES, 1), lane, jnp.int32)
    return lax.gather(v, gi, _GDN, (1,),
                      mode=lax.GatherScatterMode.PROMISE_IN_BOUNDS)


def _sc_body(coords_hbm, ia_hbm, ic_hbm, W_hbm, b_hbm,
             atab_hbm, ctab_hbm, out_hbm,
             idxa_v, idxc_v, coords_v,
             atab_v, ctab_v, out0, out1, W_v, b_v,
             so0, so1, n_tok):
    pw = n_tok // _NW
    nch = pw // _T
    wid = lax.axis_index("s") * _NC + lax.axis_index("c")
    base = wid * pw

    outs = [out0, out1]
    osems = [so0, so1]

    pro = [
        pltpu.async_copy(W_hbm, W_v, so0),
        pltpu.async_copy(b_hbm, b_v, so0),
        pltpu.async_copy(atab_hbm, atab_v, so0),
        pltpu.async_copy(ctab_hbm, ctab_v, so0),
        pltpu.async_copy(ia_hbm.at[pl.ds(base, pw)], idxa_v, so0),
        pltpu.async_copy(ic_hbm.at[pl.ds(base, pw)], idxc_v, so0),
        pltpu.async_copy(coords_hbm.at[pl.ds(base * 3, pw * 3)],
                         coords_v.at[pl.ds(0, pw * 3)], so0),
    ]
    for c in pro:
        c.wait()
    Wc = [[W_v[pl.ds(c * _D + 16 * k, 16)] for k in range(8)] for c in range(3)]
    bc = [b_v[pl.ds(16 * k, 16)] for k in range(8)]

    def pair_body(cp, carry):
        for b2 in (0, 1):
            ci = cp * 2 + b2
            ct = ci * _T
            cb = base + ct
            cbase3 = ct * 3

            @pl.when(cp > 0)
            def _wait_out():
                pltpu.make_async_copy(outs[b2],
                                      out_hbm.at[pl.ds(cb, _T)],
                                      osems[b2]).wait()

            ov = outs[b2]

            @plsc.parallel_loop(0, _T, 1, unroll=2)
            def proj_body(t):
                v = coords_v[pl.ds(cbase3 + 3 * t, _LANES)]
                bx = _bcast_lane(v, 0)
                by = _bcast_lane(v, 1)
                bz = _bcast_lane(v, 2)
                for k in range(8):
                    pr = (bc[k] + bx * Wc[0][k] + by * Wc[1][k]
                          + bz * Wc[2][k])
                    h = pr / (1.0 + jnp.exp(-pr))
                    ov[t, pl.ds(16 * k, 16)] = h

            @plsc.parallel_loop(0, _T // _G, 1, unroll=1)
            def rows_body(g):
                g16 = ct + g * _G
                iva = idxa_v[pl.ds(g16, _G)] * _D
                ivc = idxc_v[pl.ds(g16, _G)] * _D
                for u in range(_G):
                    t = g * _G + u
                    ab = iva[u]
                    cbx = ivc[u]
                    la = [atab_v[pl.ds(ab + 16 * k, 16)] for k in range(8)]
                    lc = [ctab_v[pl.ds(cbx + 16 * k, 16)] for k in range(8)]
                    for k in range(8):
                        plsc.addupdate(ov.at[t, pl.ds(16 * k, 16)],
                                       la[k] + lc[k])

            pltpu.async_copy(ov, out_hbm.at[pl.ds(cb, _T)], osems[b2])
        return carry

    lax.fori_loop(0, nch // 2, pair_body, 0)

    for b2 in (0, 1):
        cb = base + (nch - 2 + b2) * _T
        pltpu.make_async_copy(outs[b2], out_hbm.at[pl.ds(cb, _T)],
                              osems[b2]).wait()


def kernel(coords, atom_types, residue_types, meta_classes, W_coord, b_coord,
           atom_table, residue_table, meta_table):
    B, L, D = coords.shape[0], coords.shape[1], W_coord.shape[1]
    N = B * L
    pw = N // _NW
    coords_f = coords.reshape(N * 3)
    ia = atom_types.reshape(N)
    ic = (residue_types * 16 + meta_classes).reshape(N)
    W_f = W_coord.reshape(3 * D)
    atab_f = atom_table.reshape(128 * D)
    ctab_f = (residue_table[:, None, :] + meta_table[None, :, :]
              ).reshape(512 * D)

    mesh = plsc.VectorSubcoreMesh(core_axis_name="c", subcore_axis_name="s",
                                  num_cores=_NC, num_subcores=_NS)
    sc_fn = pl.kernel(
        functools.partial(_sc_body, n_tok=N),
        out_type=jax.ShapeDtypeStruct((N, _D), jnp.float32),
        mesh=mesh,
        scratch_types=[
            pltpu.VMEM((pw,), jnp.int32),
            pltpu.VMEM((pw,), jnp.int32),
            pltpu.VMEM((pw * 3 + _LANES,), jnp.float32),
            pltpu.VMEM((128 * _D,), jnp.float32),
            pltpu.VMEM((512 * _D,), jnp.float32),
            pltpu.VMEM((_T, _D), jnp.float32),
            pltpu.VMEM((_T, _D), jnp.float32),
            pltpu.VMEM((3 * _D,), jnp.float32),
            pltpu.VMEM((_D,), jnp.float32),
            pltpu.SemaphoreType.DMA,
            pltpu.SemaphoreType.DMA,
        ],
    )
    out = sc_fn(coords_f, ia, ic, W_f, b_coord, atab_f, ctab_f)
    return out.reshape(B, L, D)


# speedup vs baseline: 3.3559x; 1.0256x over previous
import functools
import jax
import jax.numpy as jnp
from jax import lax
from jax.experimental import pallas as pl
from jax.experimental.pallas import tpu as pltpu
from jax.experimental.pallas import tpu_sc as plsc

_NC, _NS, _LANES = 2, 16, 16
_NW = _NC * _NS
_D = 128
_T = 128
_G = 16

_GDN = lax.GatherDimensionNumbers(offset_dims=(), collapsed_slice_dims=(0,),
                                  start_index_map=(0,))


def _bcast_lane(v, lane):
    gi = jnp.full((_LANES, 1), lane, jnp.int32)
    return lax.gather(v, gi, _GDN, (1,),
                      mode=lax.GatherScatterMode.PROMISE_IN_BOUNDS)


def _sc_body(coords_hbm, ia_hbm, ic_hbm, W_hbm, b_hbm,
             atab_hbm, ctab_hbm, out_hbm,
             idxa_v, idxc_v, coords_v,
             atab_v, ctab_v, out0, out1, W_v, b_v,
             so0, so1, n_tok):
    pw = n_tok // _NW
    nch = pw // _T
    wid = lax.axis_index("s") * _NC + lax.axis_index("c")
    base = wid * pw

    outs = [out0, out1]
    osems = [so0, so1]

    pro = [
        pltpu.async_copy(W_hbm, W_v, so0),
        pltpu.async_copy(b_hbm, b_v, so0),
        pltpu.async_copy(atab_hbm, atab_v, so0),
        pltpu.async_copy(ctab_hbm, ctab_v, so0),
        pltpu.async_copy(ia_hbm.at[pl.ds(base, pw)],
                         idxa_v.at[pl.ds(0, pw)], so0),
        pltpu.async_copy(ic_hbm.at[pl.ds(base, pw)],
                         idxc_v.at[pl.ds(0, pw)], so0),
        pltpu.async_copy(coords_hbm.at[pl.ds(base * 3, pw * 3)],
                         coords_v.at[pl.ds(0, pw * 3)], so0),
    ]
    for c in pro:
        c.wait()
    Wc = [[W_v[pl.ds(c * _D + 16 * k, 16)] for k in range(8)] for c in range(3)]
    bc = [b_v[pl.ds(16 * k, 16)] for k in range(8)]

    def pair_body(cp, carry):
        for b2 in (0, 1):
            ci = cp * 2 + b2
            ct = ci * _T
            cb = base + ct
            cbase3 = ct * 3

            @pl.when(cp > 0)
            def _wait_out():
                pltpu.make_async_copy(outs[b2],
                                      out_hbm.at[pl.ds(cb, _T)],
                                      osems[b2]).wait()

            ov = outs[b2]

            @plsc.parallel_loop(0, _T, 1, unroll=2)
            def proj_body(t):
                v = coords_v[pl.ds(cbase3 + 3 * t, _LANES)]
                bx = _bcast_lane(v, 0)
                by = _bcast_lane(v, 1)
                bz = _bcast_lane(v, 2)
                for k in range(8):
                    pr = (bc[k] + bx * Wc[0][k] + by * Wc[1][k]
                          + bz * Wc[2][k])
                    h = pr / (1.0 + jnp.exp(-pr))
                    ov[t, pl.ds(16 * k, 16)] = h

            @plsc.parallel_loop(0, _T, 1, unroll=2)
            def rows_body(t):
                va = idxa_v[pl.ds(ct + t, _LANES)]
                vc = idxc_v[pl.ds(ct + t, _LANES)]
                ab = va[0] * _D
                cbx = vc[0] * _D
                la = [atab_v[pl.ds(ab + 16 * k, 16)] for k in range(8)]
                lc = [ctab_v[pl.ds(cbx + 16 * k, 16)] for k in range(8)]
                for k in range(8):
                    plsc.addupdate(ov.at[t, pl.ds(16 * k, 16)],
                                   la[k] + lc[k])

            pltpu.async_copy(ov, out_hbm.at[pl.ds(cb, _T)], osems[b2])
        return carry

    lax.fori_loop(0, nch // 2, pair_body, 0)

    for b2 in (0, 1):
        cb = base + (nch - 2 + b2) * _T
        pltpu.make_async_copy(outs[b2], out_hbm.at[pl.ds(cb, _T)],
                              osems[b2]).wait()


def kernel(coords, atom_types, residue_types, meta_classes, W_coord, b_coord,
           atom_table, residue_table, meta_table):
    B, L, D = coords.shape[0], coords.shape[1], W_coord.shape[1]
    N = B * L
    pw = N // _NW
    coords_f = coords.reshape(N * 3)
    ia = atom_types.reshape(N)
    ic = (residue_types * 16 + meta_classes).reshape(N)
    W_f = W_coord.reshape(3 * D)
    atab_f = atom_table.reshape(128 * D)
    ctab_f = (residue_table[:, None, :] + meta_table[None, :, :]
              ).reshape(512 * D)

    mesh = plsc.VectorSubcoreMesh(core_axis_name="c", subcore_axis_name="s",
                                  num_cores=_NC, num_subcores=_NS)
    sc_fn = pl.kernel(
        functools.partial(_sc_body, n_tok=N),
        out_type=jax.ShapeDtypeStruct((N, _D), jnp.float32),
        mesh=mesh,
        scratch_types=[
            pltpu.VMEM((pw + _LANES,), jnp.int32),
            pltpu.VMEM((pw + _LANES,), jnp.int32),
            pltpu.VMEM((pw * 3 + _LANES,), jnp.float32),
            pltpu.VMEM((128 * _D,), jnp.float32),
            pltpu.VMEM((512 * _D,), jnp.float32),
            pltpu.VMEM((_T, _D), jnp.float32),
            pltpu.VMEM((_T, _D), jnp.float32),
            pltpu.VMEM((3 * _D,), jnp.float32),
            pltpu.VMEM((_D,), jnp.float32),
            pltpu.SemaphoreType.DMA,
            pltpu.SemaphoreType.DMA,
        ],
    )
    out = sc_fn(coords_f, ia, ic, W_f, b_coord, atab_f, ctab_f)
    return out.reshape(B, L, D)
